# Initial kernel scaffold; baseline (speedup 1.0000x reference)
import jax
import jax.numpy as jnp
from jax.experimental import pallas as pl


def kernel(x, conv_w, conv_b, res0_w3, res0_b3, res0_w1, res0_b1, res1_w3, res1_b3, res1_w1, res1_b1, up0_w, up0_b, up1_w, up1_b):
    raise NotImplementedError("write your pallas kernel here")


# speedup vs baseline: 2.5165x; 2.5165x over previous
import math
from functools import partial

import jax
import jax.numpy as jnp
from jax import lax
from jax.experimental import pallas as pl
from jax.experimental.pallas import tpu as pltpu

_LANE = 128
_BF = jnp.bfloat16
_F32 = jnp.float32


def _ru(x, m):
    return (x + m - 1) // m * m


def _geom(h, w):
    w2 = w + 2
    l_out = h * w2
    l_out_p = _ru(max(l_out, _LANE), _LANE)
    l_in_p = _ru(2 * w2 + 2 + l_out_p, _LANE)
    return w2, l_out, l_out_p, l_in_p


_UP1_TAB = {
    0: ((1, -1, 3), (0, 0, 1)),
    1: ((0, 0, 2), (1, 0, 0)),
    2: ((0, 0, 3), (1, 0, 1)),
    3: ((1, 0, 2), (0, 1, 0)),
}


def _decoder_kernel(xf, cw, cb, r0w3, r0b3, r0w1, r0b1, r1w3, r1b3, r1w1,
                    r1b1, u0w, u0b, u1w, u1b, o_ref, pada, padb, ph, *,
                    w, w2, l_out, l_out_p, l_in_p):
    shift = w2 + 1
    tail = l_in_p - shift - l_out_p

    def mask(rows):
        lane = lax.broadcasted_iota(jnp.int32, (rows, l_out_p), 1)
        return (lane % w2 < w) & (lane < l_out)

    m128 = mask(128)
    m64 = mask(64)

    def padded(y, m, relu):
        if relu:
            y = jnp.maximum(y, 0.0)
        yb = jnp.where(m, y, 0.0).astype(_BF)
        rows = yb.shape[0]
        return jnp.concatenate(
            [jnp.zeros((rows, shift), _BF), yb, jnp.zeros((rows, tail), _BF)],
            axis=1)

    def conv9(x, w_ref):
        acc = None
        for t in range(9):
            off = (t // 3) * w2 + (t % 3)
            d = jnp.dot(w_ref[t], x[:, off:off + l_out_p],
                        preferred_element_type=_F32)
            acc = d if acc is None else acc + d
        return acc

    acc = conv9(xf[...], cw)
    pada[...] = padded(acc + cb[...], m128, relu=False)

    for w3, b3, w1, b1, src, dst, frelu in (
            (r0w3, r0b3, r0w1, r0b1, pada, padb, False),
            (r1w3, r1b3, r1w1, r1b1, padb, pada, True)):
        xp = src[...]
        acc3 = conv9(jnp.maximum(xp, 0), w3)
        tmid = jnp.maximum(acc3 + b3[...], 0.0).astype(_BF)
        y = jnp.dot(w1[...], tmid, preferred_element_type=_F32) + b1[...]
        y = y + xp[:, shift:shift + l_out_p].astype(_F32)
        dst[...] = padded(y, m128, relu=frelu)

    x2 = pada[...]
    for p in range(4):
        ry, rx = p // 2, p % 2
        acc = None
        for a in range(2):
            for b in range(2):
                off = (ry + a) * w2 + (rx + b)
                d = jnp.dot(u0w[p * 4 + a * 2 + b], x2[:, off:off + l_out_p],
                            preferred_element_type=_F32)
                acc = d if acc is None else acc + d
        ph[p] = padded(acc + u0b[...], m64, relu=True)

    for qy in range(4):
        for qx in range(4):
            acc = None
            for ry_, sy, kh in _UP1_TAB[qy]:
                for rx_, sx, kw in _UP1_TAB[qx]:
                    off = (1 + sy) * w2 + (1 + sx)
                    src = ph[ry_ * 2 + rx_, :, off:off + l_out_p]
                    d = jnp.dot(u1w[kh * 4 + kw], src,
                                preferred_element_type=_F32)
                    acc = d if acc is None else acc + d
            o_ref[qy * 4 + qx] = acc + u1b[...]


def kernel(x, conv_w, conv_b, res0_w3, res0_b3, res0_w1, res0_b1,
           res1_w3, res1_b3, res1_w1, res1_b1, up0_w, up0_b, up1_w, up1_b):
    n, c, h, w = x.shape
    ch = conv_w.shape[0]
    crh = res0_w3.shape[0]
    cu = up0_w.shape[1]
    w2, l_out, l_out_p, l_in_p = _geom(h, w)

    xc = jnp.transpose(x, (1, 0, 2, 3))
    xp = jnp.pad(xc, ((0, 0), (0, 0), (1, 1), (1, 1)))
    xf = xp.reshape(c, n, (h + 2) * (w + 2))
    xf = jnp.pad(xf, ((0, 0), (0, 0), (0, l_in_p - (h + 2) * (w + 2))))
    xf = xf.reshape(c, n * l_in_p).astype(_BF)

    cw = jnp.transpose(conv_w, (2, 3, 0, 1)).reshape(9, ch, c).astype(_BF)
    cb = conv_b.reshape(ch, 1)

    def res_w(w3, b3, w1, b1):
        return (jnp.transpose(w3, (2, 3, 0, 1)).reshape(9, crh, ch).astype(_BF),
                b3.reshape(crh, 1),
                w1.reshape(ch, crh).astype(_BF),
                b1.reshape(ch, 1))

    r0 = res_w(res0_w3, res0_b3, res0_w1, res0_b1)
    r1 = res_w(res1_w3, res1_b3, res1_w1, res1_b1)

    wt0 = jnp.transpose(up0_w, (1, 0, 2, 3))
    u0 = jnp.stack([wt0[:, :, 3 - 2 * a - ry, 3 - 2 * b - rx]
                    for ry in (0, 1) for rx in (0, 1)
                    for a in (0, 1) for b in (0, 1)], axis=0).astype(_BF)
    u0b = up0_b.reshape(cu, 1)

    co = up1_w.shape[1]
    wt1 = jnp.transpose(up1_w, (1, 0, 2, 3))
    wt1 = jnp.pad(wt1, ((0, 8 - co), (0, 0), (0, 0), (0, 0)))
    u1 = jnp.transpose(wt1, (2, 3, 0, 1)).reshape(16, 8, cu).astype(_BF)
    u1b = jnp.pad(up1_b, (0, 8 - co)).reshape(8, 1)

    cparams = pltpu.CompilerParams(
        dimension_semantics=("parallel",),
        vmem_limit_bytes=64 * 1024 * 1024)
    flops_img = 2 * l_out * (9 * ch * c + 2 * (9 * crh * ch + ch * crh)
                             + 16 * cu * ch + 64 * 8 * cu)
    cost = pl.CostEstimate(flops=n * flops_img, transcendentals=0,
                           bytes_accessed=2 * n * c * l_in_p
                           + 4 * n * 16 * 8 * l_out_p)

    const = lambda i: (0, 0)
    const3 = lambda i: (0, 0, 0)
    out2d = pl.pallas_call(
        partial(_decoder_kernel, w=w, w2=w2, l_out=l_out, l_out_p=l_out_p,
                l_in_p=l_in_p),
        out_shape=jax.ShapeDtypeStruct((16, 8, n * l_out_p), _F32),
        grid=(n,),
        in_specs=[
            pl.BlockSpec((c, l_in_p), lambda i: (0, i)),
            pl.BlockSpec((9, ch, c), const3),
            pl.BlockSpec((ch, 1), const),
            pl.BlockSpec((9, crh, ch), const3),
            pl.BlockSpec((crh, 1), const),
            pl.BlockSpec((ch, crh), const),
            pl.BlockSpec((ch, 1), const),
            pl.BlockSpec((9, crh, ch), const3),
            pl.BlockSpec((crh, 1), const),
            pl.BlockSpec((ch, crh), const),
            pl.BlockSpec((ch, 1), const),
            pl.BlockSpec((16, cu, ch), const3),
            pl.BlockSpec((cu, 1), const),
            pl.BlockSpec((16, 8, cu), const3),
            pl.BlockSpec((8, 1), const),
        ],
        out_specs=pl.BlockSpec((16, 8, l_out_p), lambda i: (0, 0, i)),
        scratch_shapes=[
            pltpu.VMEM((ch, l_in_p), _BF),
            pltpu.VMEM((ch, l_in_p), _BF),
            pltpu.VMEM((4, cu, l_in_p), _BF),
        ],
        compiler_params=cparams,
        cost_estimate=cost,
    )(xf, cw, cb, *r0, *r1, u0, u0b, u1, u1b)

    o = out2d.reshape(4, 4, 8, n, l_out_p)[..., :l_out]
    o = o.reshape(4, 4, 8, n, h, w2)[..., :w]
    o = o[:, :, :co]
    o = jnp.transpose(o, (3, 2, 4, 0, 5, 1)).reshape(n, co, 4 * h, 4 * w)
    return o


# speedup vs baseline: 3.4564x; 1.3735x over previous
from functools import partial

import jax
import jax.numpy as jnp
from jax import lax
from jax.experimental import pallas as pl
from jax.experimental.pallas import tpu as pltpu

_LANE = 128
_BF = jnp.bfloat16
_F32 = jnp.float32

_SRC = ((1, -1), (0, 0), (1, 0), (0, 1))
_READERS = (((0, 3),), ((0, 1), (1, 2), (2, 3)), ((1, 0), (2, 1), (3, 2)),
            ((3, 0),))


def _ru(x, m):
    return (x + m - 1) // m * m


def _geom(h, w):
    w2 = w + 2
    l_out = h * w2
    l_out_p = _ru(max(l_out, _LANE), _LANE)
    l_in_p = _ru(2 * w2 + 2 + l_out_p, _LANE)
    return w2, l_out, l_out_p, l_in_p


def _decoder_kernel(xf, cw, cb, r0w3, r0b3, r0w1, r0b1, r1w3, r1b3, r1w1,
                    r1b1, u0l, u0b, wbig, bout, o_ref,
                    s1, s2a, s2b, s3, s4, hres, *,
                    c, ch, cu, w, w2, l_out, l_out_p, l_in_p):
    shift = w2 + 1
    tail = l_in_p - shift - l_out_p

    def mask(rows):
        lane = lax.broadcasted_iota(jnp.int32, (rows, l_out_p), 1)
        return (lane % w2 < w) & (lane < l_out)

    m_ch = mask(ch)
    m_cu = mask(cu)

    def padded(y, m):
        yb = jnp.where(m, y, 0.0).astype(_BF)
        rows = yb.shape[0]
        return jnp.concatenate(
            [jnp.zeros((rows, shift), _BF), yb, jnp.zeros((rows, tail), _BF)],
            axis=1)

    def stack9(dst, f, rows):
        for t in range(9):
            off = (t // 3) * w2 + (t % 3)
            dst[t * rows:(t + 1) * rows, :] = f[:, off:off + l_out_p]

    stack9(s1, xf[...], c)
    h = jnp.dot(cw[...], s1[...], preferred_element_type=_F32) + cb[...]

    for w3, b3, w1, b1, s2 in ((r0w3, r0b3, r0w1, r0b1, s2a),
                               (r1w3, r1b3, r1w1, r1b1, s2b)):
        hres[...] = h.astype(_BF)
        stack9(s2, padded(jnp.maximum(h, 0.0), m_ch), ch)
        t3 = jnp.dot(w3[...], s2[...], preferred_element_type=_F32) + b3[...]
        t3 = jnp.maximum(t3, 0.0).astype(_BF)
        h = jnp.dot(w1[...], t3, preferred_element_type=_F32) + b1[...]
        h = h + hres[...].astype(_F32)

    stack9(s3, padded(jnp.maximum(h, 0.0), m_ch), ch)
    for p in range(4):
        ry, rx = p // 2, p % 2
        r0 = (ry * 3 + rx) * ch
        acc = jnp.dot(u0l[2 * p], s3[r0:r0 + 2 * ch],
                      preferred_element_type=_F32)
        acc = acc + jnp.dot(u0l[2 * p + 1], s3[r0 + 3 * ch:r0 + 5 * ch],
                            preferred_element_type=_F32)
        fp = padded(jnp.maximum(acc + u0b[...], 0.0), m_cu)
        for ya in range(4):
            for xb in range(4):
                if (_SRC[ya][0] * 2 + _SRC[xb][0]) != p:
                    continue
                off = (1 + _SRC[ya][1]) * w2 + (1 + _SRC[xb][1])
                s = ya * 4 + xb
                s4[s * cu:(s + 1) * cu, :] = fp[:, off:off + l_out_p]

    o = jnp.dot(wbig[...], s4[...], preferred_element_type=_F32) + bout[...]
    o_ref[...] = o.astype(_BF)


def kernel(x, conv_w, conv_b, res0_w3, res0_b3, res0_w1, res0_b1,
           res1_w3, res1_b3, res1_w1, res1_b1, up0_w, up0_b, up1_w, up1_b):
    n, c, h, w = x.shape
    ch = conv_w.shape[0]
    crh = res0_w3.shape[0]
    cu = up0_w.shape[1]
    co = up1_w.shape[1]
    w2, l_out, l_out_p, l_in_p = _geom(h, w)

    xc = jnp.transpose(x, (1, 0, 2, 3)).astype(_BF)
    xp = jnp.pad(xc, ((0, 0), (0, 0), (1, 1), (1, 1)))
    xf = xp.reshape(c, n, (h + 2) * (w + 2))
    xf = jnp.pad(xf, ((0, 0), (0, 0), (0, l_in_p - (h + 2) * (w + 2))))
    xf = xf.reshape(c, n * l_in_p)

    cw = jnp.transpose(conv_w, (0, 2, 3, 1)).reshape(ch, 9 * c).astype(_BF)
    cb = conv_b.reshape(ch, 1)

    def res_w(w3, b3, w1, b1):
        return (jnp.transpose(w3, (0, 2, 3, 1)).reshape(crh, 9 * ch).astype(_BF),
                b3.reshape(crh, 1),
                w1.reshape(ch, crh).astype(_BF),
                b1.reshape(ch, 1))

    r0 = res_w(res0_w3, res0_b3, res0_w1, res0_b1)
    r1 = res_w(res1_w3, res1_b3, res1_w1, res1_b1)

    wt0 = jnp.transpose(up0_w, (1, 0, 2, 3))
    u0l = jnp.stack([
        jnp.concatenate([wt0[:, :, 3 - 2 * a - ry, 3 - rx],
                         wt0[:, :, 3 - 2 * a - ry, 1 - rx]], axis=1)
        for ry in (0, 1) for rx in (0, 1) for a in (0, 1)], axis=0).astype(_BF)
    u0b = up0_b.reshape(cu, 1)

    wt1 = jnp.transpose(up1_w, (1, 0, 2, 3))
    wt1 = jnp.pad(wt1, ((0, 8 - co), (0, 0), (0, 0), (0, 0)))
    zero8 = jnp.zeros((8, cu), _F32)
    rows = []
    for qy in range(4):
        for qx in range(4):
            blocks = []
            for ya in range(4):
                khm = dict(_READERS[ya])
                for xb in range(4):
                    kwm = dict(_READERS[xb])
                    if qy in khm and qx in kwm:
                        blocks.append(wt1[:, :, khm[qy], kwm[qx]])
                    else:
                        blocks.append(zero8)
            rows.append(jnp.concatenate(blocks, axis=1))
    wbig = jnp.concatenate(rows, axis=0).astype(_BF)
    bout = jnp.tile(jnp.pad(up1_b, (0, 8 - co)), 16).reshape(16 * 8, 1)

    cparams = pltpu.CompilerParams(
        dimension_semantics=("parallel",),
        vmem_limit_bytes=64 * 1024 * 1024)
    flops_img = 2 * l_out * (9 * ch * c + 2 * (9 * crh * ch + ch * crh)
                             + 16 * cu * ch + 128 * 16 * cu)
    cost = pl.CostEstimate(flops=n * flops_img, transcendentals=0,
                           bytes_accessed=2 * n * c * l_in_p
                           + 2 * n * 16 * 8 * l_out_p)

    const = lambda i: (0, 0)
    const3 = lambda i: (0, 0, 0)
    out2d = pl.pallas_call(
        partial(_decoder_kernel, c=c, ch=ch, cu=cu, w=w, w2=w2, l_out=l_out,
                l_out_p=l_out_p, l_in_p=l_in_p),
        out_shape=jax.ShapeDtypeStruct((16 * 8, n * l_out_p), _BF),
        grid=(n,),
        in_specs=[
            pl.BlockSpec((c, l_in_p), lambda i: (0, i)),
            pl.BlockSpec((ch, 9 * c), const),
            pl.BlockSpec((ch, 1), const),
            pl.BlockSpec((crh, 9 * ch), const),
            pl.BlockSpec((crh, 1), const),
            pl.BlockSpec((ch, crh), const),
            pl.BlockSpec((ch, 1), const),
            pl.BlockSpec((crh, 9 * ch), const),
            pl.BlockSpec((crh, 1), const),
            pl.BlockSpec((ch, crh), const),
            pl.BlockSpec((ch, 1), const),
            pl.BlockSpec((8, cu, 2 * ch), const3),
            pl.BlockSpec((cu, 1), const),
            pl.BlockSpec((16 * 8, 16 * cu), const),
            pl.BlockSpec((16 * 8, 1), const),
        ],
        out_specs=pl.BlockSpec((16 * 8, l_out_p), lambda i: (0, i)),
        scratch_shapes=[
            pltpu.VMEM((9 * c, l_out_p), _BF),
            pltpu.VMEM((9 * ch, l_out_p), _BF),
            pltpu.VMEM((9 * ch, l_out_p), _BF),
            pltpu.VMEM((9 * ch, l_out_p), _BF),
            pltpu.VMEM((16 * cu, l_out_p), _BF),
            pltpu.VMEM((ch, l_out_p), _BF),
        ],
        compiler_params=cparams,
        cost_estimate=cost,
    )(xf, cw, cb, *r0, *r1, u0l, u0b, wbig, bout)

    o = out2d.reshape(16, 8, n, l_out_p)[..., :l_out]
    o = o.reshape(16, 8, n, h, w2)[..., :w]
    o = o[:, :co].reshape(4, 4, co, n, h, w)
    o = jnp.transpose(o, (3, 2, 4, 0, 5, 1)).reshape(n, co, 4 * h, 4 * w)
    return o.astype(_F32)
